# Initial kernel scaffold; baseline (speedup 1.0000x reference)
import jax
import jax.numpy as jnp
from jax.experimental import pallas as pl


def kernel(pos, atom_type_onehot, bond_type_onehot, ring_table_node, Wa_node, Wlin_node, blin_node, rbf_mean, rbf_std, rbf_w, rbf_b, Wb, W0f, W1f, W2f, rad_L1, rad_b1, rad_L2, rad_b2, rad_L3, rad_b3, ring_table_ed, Wa_ed, Wp0, bp0, Wp2, ring_info, edge_index, batch):
    raise NotImplementedError("write your pallas kernel here")


# speedup vs baseline: 1.3592x; 1.3592x over previous
import functools
import math

import jax
import jax.numpy as jnp
from jax import lax
from jax.experimental import pallas as pl
from jax.experimental.pallas import tpu as pltpu
from jax.experimental.pallas import tpu_sc as plsc

N = 10000
E = 160000
H = 32
NB = 128
BD = 32
M0, M1, M2 = 64, 32, 16
DIM = M0 + 3 * M1 + 5 * M2
CUTOFF = 1000.0
SCW = NB + BD

BE = 2000
BN = 2000
NBLK_E = E // BE
NBLK_N = N // BN

NW = 32
_C = 128
_EW = E // NW
_NCH = _EW // _C
_REM = _EW - _NCH * _C
_RPT = N // 16

_PREC = lax.Precision.HIGHEST


def _silu(x):
    return x * (1.0 / (1.0 + jnp.exp(-x)))


def _expander(k, m):
    j = lax.broadcasted_iota(jnp.int32, (k, k * m), 1)
    a = lax.broadcasted_iota(jnp.int32, (k, k * m), 0)
    r = (j // m == a).astype(jnp.float32)
    jb = lax.broadcasted_iota(jnp.int32, (m, k * m), 1)
    b = lax.broadcasted_iota(jnp.int32, (m, k * m), 0)
    t = (jb % m == b).astype(jnp.float32)
    return r, t


def _prep_body(pos, ringf, onehot, tbl_ed, wa_ed, t8, tnf):
    p = pos[...]
    t8[...] = jnp.concatenate(
        [p, jnp.zeros((p.shape[0], 5), jnp.float32)], axis=1)
    t0 = tbl_ed[...][0:1, :]
    t1 = tbl_ed[...][1:2, :]
    tnf[...] = t0 + ringf[...] * (t1 - t0) + jnp.dot(
        onehot[...], wa_ed[...], precision=_PREC)


def _prep_call(pos, ringf, onehot, tbl_ed, wa_ed, interpret=False):
    return pl.pallas_call(
        _prep_body,
        out_shape=(jax.ShapeDtypeStruct((N, 8), jnp.float32),
                   jax.ShapeDtypeStruct((N, H), jnp.float32)),
        interpret=interpret,
    )(pos, ringf, onehot, tbl_ed, wa_ed)


def _gather_call(tbl8, tblnf, src, dst):
    mesh = plsc.VectorSubcoreMesh(core_axis_name="c", subcore_axis_name="s")

    @functools.partial(
        pl.kernel, mesh=mesh,
        compiler_params=pltpu.CompilerParams(use_tc_tiling_on_sc=False),
        out_type=(
            jax.ShapeDtypeStruct((E, H), jnp.float32),
            jax.ShapeDtypeStruct((E, 8), jnp.float32),
            jax.ShapeDtypeStruct((E, 8), jnp.float32),
        ),
        scratch_types=[
            pltpu.VMEM((_C,), jnp.int32),
            pltpu.VMEM((_C,), jnp.int32),
            pltpu.VMEM((_REM,), jnp.int32),
            pltpu.VMEM((_REM,), jnp.int32),
            pltpu.VMEM((_C, H), jnp.float32),
            pltpu.VMEM((_C, 8), jnp.float32),
            pltpu.VMEM((_C, 8), jnp.float32),
            pltpu.VMEM((_REM, H), jnp.float32),
            pltpu.VMEM((_REM, 8), jnp.float32),
            pltpu.VMEM((_REM, 8), jnp.float32),
            pltpu.SemaphoreType.DMA,
            pltpu.SemaphoreType.DMA,
            pltpu.SemaphoreType.DMA,
        ],
    )
    def k(tbl8_h, tblnf_h, src_h, dst_h, ose_h, osrc_h, odst_h,
          is_v, id_v, is_r, id_r, se_v, s8_v, d8_v, se_r, s8_r, d8_r,
          sem1, sem2, sem3):
        c = lax.axis_index("c")
        s = lax.axis_index("s")
        wid = s * 2 + c
        base0 = wid * _EW

        def do_chunk(base, n, isv, idv, sev, s8v, d8v):
            bs = pl.multiple_of(base, 8)
            pltpu.sync_copy(src_h.at[pl.ds(bs, n)], isv)
            pltpu.sync_copy(dst_h.at[pl.ds(bs, n)], idv)
            cp1 = pltpu.async_copy(tblnf_h.at[isv], sev, sem1)
            cp2 = pltpu.async_copy(tbl8_h.at[isv], s8v, sem2)
            cp3 = pltpu.async_copy(tbl8_h.at[idv], d8v, sem3)
            cp1.wait()
            cp2.wait()
            cp3.wait()
            pltpu.sync_copy(sev, ose_h.at[pl.ds(bs, n)])
            pltpu.sync_copy(s8v, osrc_h.at[pl.ds(bs, n)])
            pltpu.sync_copy(d8v, odst_h.at[pl.ds(bs, n)])

        def body(j, carry):
            do_chunk(base0 + j * _C, _C, is_v, id_v, se_v, s8_v, d8_v)
            return carry

        lax.fori_loop(0, _NCH, body, 0)
        do_chunk(base0 + _NCH * _C, _REM, is_r, id_r, se_r, s8_r, d8_r)

    return k(tbl8, tblnf, src, dst)


def _edge_body(gse, gsrc8, gdst8, bond, rbm, rbs, rbw, rbb, wb,
               w0f, w1f, w2f, l1, b1, l2, b2, l3, b3, wp2, outf, outp):
    se = gse[...]
    ps = gsrc8[...][:, 0:3]
    pd = gdst8[...][:, 0:3]
    evec = ps - pd
    r = jnp.sqrt(jnp.sum(evec * evec, axis=1, keepdims=True))
    u = evec / (r + 1e-9)
    ux = u[:, 0:1]
    uy = u[:, 1:2]
    uz = u[:, 2:3]
    s1 = jnp.sqrt(3.0) * jnp.concatenate([uy, uz, ux], axis=1)
    c15 = jnp.sqrt(15.0)
    s2 = jnp.concatenate([
        c15 * ux * uy,
        c15 * uy * uz,
        (jnp.sqrt(5.0) / 2.0) * (3.0 * uz * uz - 1.0),
        c15 * ux * uz,
        (c15 / 2.0) * (ux * ux - uy * uy),
    ], axis=1)
    xs = r / CUTOFF
    v = rbw[...] * xs + rbb[...]
    stdp = jnp.abs(rbs[...]) + 1e-5
    ainv = 1.0 / ((2.0 * 3.14159) ** 0.5)
    t = (v - rbm[...]) / stdp
    g = jnp.exp(-0.5 * t * t) * (ainv / stdp)
    bnd = jnp.dot(bond[...], wb[...], precision=_PREC)
    scal = jnp.concatenate([g, bnd], axis=1)
    inv = 1.0 / math.sqrt(float(SCW))
    c0 = jnp.dot(scal, w0f[...], precision=_PREC) * inv
    c1 = jnp.dot(scal, w1f[...], precision=_PREC) * inv
    c2 = jnp.dot(scal, w2f[...], precision=_PREC) * inv
    r1, t1m = _expander(M1, 3)
    r2, t2m = _expander(M2, 5)
    st1 = jnp.dot(s1, t1m, precision=_PREC)
    st2 = jnp.dot(s2, t2m, precision=_PREC)
    f1 = jnp.dot(c1, r1, precision=_PREC) * st1
    f2 = jnp.dot(c2, r2, precision=_PREC) * st2
    outf[...] = jnp.concatenate([c0, f1, f2], axis=1)
    h = _silu(jnp.dot(g, l1[...], precision=_PREC) + b1[...])
    h = _silu(jnp.dot(h, l2[...], precision=_PREC) + b2[...])
    w = jnp.dot(h, l3[...], precision=_PREC) + b3[...]
    w0 = w[:, 0:H]
    w2 = w[:, H:2 * H]
    d0 = se * w0
    bvec = jnp.dot(se * w2, wp2[...], precision=_PREC) * (1.0 / math.sqrt(H))
    p2 = jnp.dot(bvec, r2, precision=_PREC) * st2
    ones = jnp.ones((d0.shape[0], 1), jnp.float32)
    zer = jnp.zeros((d0.shape[0], 15), jnp.float32)
    outp[...] = jnp.concatenate([d0, p2, ones, zer], axis=1)


def _edge_call(gse, gsrc8, gdst8, bond, rbm, rbs, rbw, rbb, wb,
               w0f, w1f, w2f, l1, b1, l2, b2, l3, b3, wp2, interpret=False):
    def full(shape):
        return pl.BlockSpec(shape, lambda i: (0,) * len(shape))

    in_specs = [
        pl.BlockSpec((BE, H), lambda i: (i, 0)),
        pl.BlockSpec((BE, 8), lambda i: (i, 0)),
        pl.BlockSpec((BE, 8), lambda i: (i, 0)),
        pl.BlockSpec((BE, 5), lambda i: (i, 0)),
        full((1, NB)), full((1, NB)), full((1, 1)), full((1, 1)),
        full((5, BD)), full((SCW, M0)), full((SCW, M1)), full((SCW, M2)),
        full((NB, 64)), full((1, 64)), full((64, 64)), full((1, 64)),
        full((64, 2 * H)), full((1, 2 * H)), full((H, M2)),
    ]
    out_specs = (
        pl.BlockSpec((BE, DIM), lambda i: (i + NBLK_N, 0)),
        pl.BlockSpec((BE, 128), lambda i: (i, 0)),
    )
    return pl.pallas_call(
        _edge_body,
        grid=(NBLK_E,),
        in_specs=in_specs,
        out_specs=out_specs,
        out_shape=(jax.ShapeDtypeStruct((N + E, DIM), jnp.float32),
                   jax.ShapeDtypeStruct((E, 128), jnp.float32)),
        interpret=interpret,
    )(gse, gsrc8, gdst8, bond, rbm, rbs, rbw, rbb, wb,
      w0f, w1f, w2f, l1, b1, l2, b2, l3, b3, wp2)


def _scatter_call(payload, dst, zrows):
    mesh = plsc.VectorSubcoreMesh(core_axis_name="c", subcore_axis_name="s")

    @functools.partial(
        pl.kernel, mesh=mesh,
        compiler_params=pltpu.CompilerParams(use_tc_tiling_on_sc=False),
        out_type=jax.ShapeDtypeStruct((2, N, 128), jnp.float32),
        scratch_types=[
            pltpu.VMEM((1, _C), jnp.int32),
            pltpu.VMEM((1, _REM), jnp.int32),
            pltpu.VMEM((_C, 128), jnp.float32),
            pltpu.VMEM((_REM, 128), jnp.float32),
            pltpu.VMEM_SHARED((N, 128), jnp.float32),
        ],
    )
    def k(pay_h, dst_h, z_h, out_h, idx_v, idx_r, pay_v, pay_r, acc_sh):
        c = lax.axis_index("c")
        s = lax.axis_index("s")
        r0 = s * _RPT
        pltpu.sync_copy(z_h, acc_sh.at[pl.ds(r0, _RPT)])
        plsc.subcore_barrier()
        base0 = c * (E // 2) + s * _EW

        def do_chunk(base, n, idxv, payv):
            bs = pl.multiple_of(base, 8)
            pltpu.sync_copy(dst_h.at[pl.ds(bs, n)], idxv.at[0])
            pltpu.sync_copy(pay_h.at[pl.ds(bs, n)], payv)
            pltpu.sync_copy(payv, acc_sh.at[idxv.at[0]], add=True)

        def body(j, carry):
            do_chunk(base0 + j * _C, _C, idx_v, pay_v)
            return carry

        lax.fori_loop(0, _NCH, body, 0)
        do_chunk(base0 + _NCH * _C, _REM, idx_r, pay_r)
        plsc.subcore_barrier()
        pltpu.sync_copy(acc_sh.at[pl.ds(r0, _RPT)],
                        out_h.at[c, pl.ds(r0, _RPT)])

    return k(payload, dst, zrows)


def _node_body(ringf, onehot, acc0, acc1, tbl, wa, wlin, blin, wp0, bp0v,
               big_in, out):
    racc = acc0[0] + acc1[0]
    d0 = racc[:, 0:H]
    p2 = racc[:, H:H + 5 * M2]
    cnt = racc[:, H + 5 * M2:H + 5 * M2 + 1]
    rf = ringf[...]
    t0 = tbl[...][0:1, :]
    t1 = tbl[...][1:2, :]
    nf = t0 + rf * (t1 - t0) + jnp.dot(onehot[...], wa[...], precision=_PREC)
    n0 = jnp.dot(nf, wlin[...], precision=_PREC) * (1.0 / math.sqrt(H)) \
        + blin[...]
    deg0 = (jnp.dot(d0, wp0[...], precision=_PREC) * (1.0 / math.sqrt(H))
            + cnt * bp0v[...]) * 0.25
    mid = jnp.zeros((n0.shape[0], 3 * M1), jnp.float32)
    out[...] = jnp.concatenate([n0 + deg0, mid, p2 * 0.25], axis=1)


def _node_call(ringf, onehot, acc, tbl, wa, wlin, blin, wp0, bp0v, bigout,
               interpret=False):
    def full(shape):
        return pl.BlockSpec(shape, lambda i: (0,) * len(shape))

    in_specs = [
        pl.BlockSpec((BN, 1), lambda i: (i, 0)),
        pl.BlockSpec((BN, 6), lambda i: (i, 0)),
        pl.BlockSpec((1, BN, 128), lambda i: (0, i, 0)),
        pl.BlockSpec((1, BN, 128), lambda i: (1, i, 0)),
        full((2, H)), full((6, H)), full((H, M0)), full((1, M0)),
        full((H, M0)), full((1, M0)),
        pl.BlockSpec((BN, DIM), lambda i: (i, 0)),
    ]
    return pl.pallas_call(
        _node_body,
        grid=(NBLK_N,),
        in_specs=in_specs,
        out_specs=pl.BlockSpec((BN, DIM), lambda i: (i, 0)),
        out_shape=jax.ShapeDtypeStruct((N + E, DIM), jnp.float32),
        input_output_aliases={10: 0},
        interpret=interpret,
    )(ringf, onehot, acc, acc, tbl, wa, wlin, blin, wp0, bp0v, bigout)


def kernel(pos, atom_type_onehot, bond_type_onehot, ring_table_node, Wa_node,
           Wlin_node, blin_node, rbf_mean, rbf_std, rbf_w, rbf_b, Wb, W0f,
           W1f, W2f, rad_L1, rad_b1, rad_L2, rad_b2, rad_L3, rad_b3,
           ring_table_ed, Wa_ed, Wp0, bp0, Wp2, ring_info, edge_index, batch):
    src = edge_index[0]
    dst = edge_index[1]
    ringf = ring_info.astype(jnp.float32).reshape(N, 1)
    b1 = rad_b1.reshape(1, 64)
    b2 = rad_b2.reshape(1, 64)
    b3 = rad_b3.reshape(1, 2 * H)
    blin = blin_node.reshape(1, M0)
    bp0v = bp0.reshape(1, M0)
    tbl8, tblnf = _prep_call(pos, ringf, atom_type_onehot,
                             ring_table_ed, Wa_ed)
    gse, gsrc8, gdst8 = _gather_call(tbl8, tblnf, src, dst)
    bigout, payload = _edge_call(
        gse, gsrc8, gdst8, bond_type_onehot, rbf_mean, rbf_std, rbf_w, rbf_b,
        Wb, W0f, W1f, W2f, rad_L1, b1, rad_L2, b2, rad_L3, b3, Wp2)
    zrows = jnp.zeros((_RPT, 128), jnp.float32)
    acc = _scatter_call(payload, dst, zrows)
    return _node_call(ringf, atom_type_onehot, acc, ring_table_node, Wa_node,
                      Wlin_node, blin, Wp0, bp0v, bigout)
